# baseline (speedup 1.0000x reference)
import functools
import jax
import jax.numpy as jnp
from jax.experimental import pallas as pl
from jax.experimental.pallas import tpu as pltpu

TOKENS = 8192
INPUT_DIM = 1024
HIDDEN_DIM = 256
POOL_SIZE = 16384
K = 1024
MIN_P = 100.0
MAX_P = 1024.0

ROW_A = 256
ROW_B = 128
ROW_D = 128
MAXI32 = 0x7FFFFFFF


def _hs_complexity_body(x_ref, W1_ref, b1_ref, W2_ref, b2_ref, S1_ref, bs1_ref,
                        hs_ref, comp_ref):
    x = x_ref[...]
    h = jnp.maximum(jnp.dot(x, W1_ref[...], preferred_element_type=jnp.float32)
                    + b1_ref[...], 0.0)
    logit = jnp.dot(h, W2_ref[...], preferred_element_type=jnp.float32) + b2_ref[...]
    comp_ref[...] = jax.nn.sigmoid(logit)
    hs_ref[...] = jnp.maximum(
        jnp.dot(x, S1_ref[...], preferred_element_type=jnp.float32) + bs1_ref[...], 0.0)


def _f32_to_key(s):
    b = jax.lax.bitcast_convert_type(s, jnp.int32)
    return jnp.where(b >= 0, b, MAXI32 - b)


def _key_to_f32(k):
    b = jnp.where(k >= 0, k, MAXI32 - k)
    return jax.lax.bitcast_convert_type(b, jnp.float32)


def _keys_bisect_body(hs_ref, S2_ref, bs2_ref, keys_ref, t_ref, quota_ref,
                      lo_ref, hi_ref, fr_ref):
    scores = (jnp.dot(hs_ref[...], S2_ref[...],
                      preferred_element_type=jnp.float32) + bs2_ref[...])
    keys = _f32_to_key(scores)
    keys_ref[...] = keys

    lo_ref[...] = jnp.min(keys, axis=1, keepdims=True)
    hi_ref[...] = jnp.max(keys, axis=1, keepdims=True) + 1
    fr_ref[...] = jnp.zeros(fr_ref.shape, jnp.int32)

    def cond(carry):
        it, done = carry
        return jnp.logical_and(it < 34, done == 0)

    def body(carry):
        it, done = carry
        lo = lo_ref[...]
        hi = hi_ref[...]
        fr = fr_ref[...]
        mid = (lo >> 1) + (hi >> 1) + (lo & hi & 1)
        cnt = jnp.sum((keys >= mid).astype(jnp.int32), axis=1, keepdims=True)
        active = (fr == 0) & (hi - lo != 1)
        ge = cnt >= K
        new_lo = jnp.where(active & ge, mid, lo)
        new_hi = jnp.where(active & jnp.logical_not(ge), mid, hi)
        new_fr = fr | (active & (cnt == K)).astype(jnp.int32)
        lo_ref[...] = new_lo
        hi_ref[...] = new_hi
        fr_ref[...] = new_fr
        all_done = jnp.all((new_fr != 0) | (new_hi - new_lo == 1))
        return it + 1, all_done.astype(jnp.int32)

    jax.lax.while_loop(cond, body, (jnp.int32(0), jnp.int32(0)))

    lo = lo_ref[...]
    frozen = fr_ref[...] != 0
    sel = keys >= lo
    minsel = jnp.min(jnp.where(sel, keys, MAXI32), axis=1, keepdims=True)
    t = jnp.where(frozen, minsel, lo)
    cnt_gt = jnp.sum((keys > t).astype(jnp.int32), axis=1, keepdims=True)
    quota = K - cnt_gt
    t_ref[...] = jnp.broadcast_to(t, t_ref.shape)
    quota_ref[...] = jnp.broadcast_to(quota, quota_ref.shape)


def _bitonic_sort_desc(keys, idx):
    R, N = keys.shape
    lane = jax.lax.broadcasted_iota(jnp.int32, (R, N), 1)
    k = 2
    while k <= N:
        j = k // 2
        while j >= 1:
            low = (lane & j) == 0
            pk = jnp.where(low, pltpu.roll(keys, N - j, 1), pltpu.roll(keys, j, 1))
            pi = jnp.where(low, pltpu.roll(idx, N - j, 1), pltpu.roll(idx, j, 1))
            self_beats = (keys > pk) | ((keys == pk) & (idx < pi))
            want_max = ((lane & k) == 0) == low
            take_self = want_max == self_beats
            keys = jnp.where(take_self, keys, pk)
            idx = jnp.where(take_self, idx, pi)
            j //= 2
        k *= 2
    return keys, idx


def _sort_finalize_body(ckeys_ref, cidx_ref, comp_ref,
                        out_idx_ref, out_w_ref, out_mask_ref):
    keys, idx = _bitonic_sort_desc(ckeys_ref[...], cidx_ref[...])
    ts = _key_to_f32(keys)
    m = ts[:, 0:1]
    e = jnp.exp(ts - m)
    w = e / jnp.sum(e, axis=1, keepdims=True)
    comp = comp_ref[...]
    budgets = jnp.round(
        jnp.clip(MIN_P + (MAX_P - MIN_P) * comp * comp, MIN_P, MAX_P)
    ).astype(jnp.int32)
    pos = jax.lax.broadcasted_iota(jnp.int32, out_mask_ref.shape, 1)
    mask = (pos < budgets).astype(jnp.float32)
    out_idx_ref[...] = idx
    out_w_ref[...] = w * mask
    out_mask_ref[...] = mask


def kernel(x, W1, b1, W2, b2, S1, bs1, S2, bs2):
    n_ra = TOKENS // ROW_A
    hs, comp = pl.pallas_call(
        _hs_complexity_body,
        grid=(n_ra,),
        in_specs=[
            pl.BlockSpec((ROW_A, INPUT_DIM), lambda i: (i, 0)),
            pl.BlockSpec((INPUT_DIM, 128), lambda i: (0, 0)),
            pl.BlockSpec((128,), lambda i: (0,)),
            pl.BlockSpec((128, 1), lambda i: (0, 0)),
            pl.BlockSpec((1,), lambda i: (0,)),
            pl.BlockSpec((INPUT_DIM, HIDDEN_DIM), lambda i: (0, 0)),
            pl.BlockSpec((HIDDEN_DIM,), lambda i: (0,)),
        ],
        out_specs=[
            pl.BlockSpec((ROW_A, HIDDEN_DIM), lambda i: (i, 0)),
            pl.BlockSpec((ROW_A, 1), lambda i: (i, 0)),
        ],
        out_shape=[
            jax.ShapeDtypeStruct((TOKENS, HIDDEN_DIM), jnp.float32),
            jax.ShapeDtypeStruct((TOKENS, 1), jnp.float32),
        ],
    )(x, W1, b1, W2, b2, S1, bs1)

    n_rb = TOKENS // ROW_B
    keys, t_b, quota_b = pl.pallas_call(
        _keys_bisect_body,
        grid=(n_rb,),
        in_specs=[
            pl.BlockSpec((ROW_B, HIDDEN_DIM), lambda i: (i, 0)),
            pl.BlockSpec((HIDDEN_DIM, POOL_SIZE), lambda i: (0, 0)),
            pl.BlockSpec((POOL_SIZE,), lambda i: (0,)),
        ],
        out_specs=[
            pl.BlockSpec((ROW_B, POOL_SIZE), lambda i: (i, 0)),
            pl.BlockSpec((ROW_B, 16), lambda i: (i, 0)),
            pl.BlockSpec((ROW_B, 16), lambda i: (i, 0)),
        ],
        out_shape=[
            jax.ShapeDtypeStruct((TOKENS, POOL_SIZE), jnp.int32),
            jax.ShapeDtypeStruct((TOKENS, 16), jnp.int32),
            jax.ShapeDtypeStruct((TOKENS, 16), jnp.int32),
        ],
        scratch_shapes=[
            pltpu.VMEM((ROW_B, 1), jnp.int32),
            pltpu.VMEM((ROW_B, 1), jnp.int32),
            pltpu.VMEM((ROW_B, 1), jnp.int32),
        ],
    )(hs, S2, bs2)

    t = t_b[:, 0:1]
    quota = quota_b[:, 0:1]
    gt = keys > t
    eq = keys == t
    eq_rank = jnp.cumsum(eq.astype(jnp.int32), axis=1)
    sel = gt | (eq & (eq_rank <= quota))
    order = jnp.argsort(jnp.logical_not(sel), axis=1, stable=True)[:, :K]
    ckeys = jnp.take_along_axis(keys, order, axis=1)
    cidx = jnp.sort(order, axis=1)
    ckeys = jnp.take_along_axis(keys, cidx, axis=1)

    n_rd = TOKENS // ROW_D
    out_idx, out_w, out_mask = pl.pallas_call(
        _sort_finalize_body,
        grid=(n_rd,),
        in_specs=[
            pl.BlockSpec((ROW_D, K), lambda i: (i, 0)),
            pl.BlockSpec((ROW_D, K), lambda i: (i, 0)),
            pl.BlockSpec((ROW_D, 1), lambda i: (i, 0)),
        ],
        out_specs=[
            pl.BlockSpec((ROW_D, K), lambda i: (i, 0)),
            pl.BlockSpec((ROW_D, K), lambda i: (i, 0)),
            pl.BlockSpec((ROW_D, K), lambda i: (i, 0)),
        ],
        out_shape=[
            jax.ShapeDtypeStruct((TOKENS, K), jnp.int32),
            jax.ShapeDtypeStruct((TOKENS, K), jnp.float32),
            jax.ShapeDtypeStruct((TOKENS, K), jnp.float32),
        ],
    )(ckeys, cidx, comp)
    return out_idx, out_w, out_mask, comp


# speedup vs baseline: 4.5631x; 4.5631x over previous
import functools
import jax
import jax.numpy as jnp
from jax import lax
from jax.experimental import pallas as pl
from jax.experimental.pallas import tpu as pltpu
from jax.experimental.pallas import tpu_sc as plsc

TOKENS = 8192
INPUT_DIM = 1024
HIDDEN_DIM = 256
POOL_SIZE = 16384
K = 1024
MIN_P = 100.0
MAX_P = 1024.0

ROW_A = 256
ROW_B = 128
ROW_D = 128
MAXI32 = 0x7FFFFFFF


def _hs_complexity_body(x_ref, W1_ref, b1_ref, W2_ref, b2_ref, S1_ref, bs1_ref,
                        hs_ref, comp_ref):
    x = x_ref[...]
    h = jnp.maximum(jnp.dot(x, W1_ref[...], preferred_element_type=jnp.float32)
                    + b1_ref[...], 0.0)
    logit = jnp.dot(h, W2_ref[...], preferred_element_type=jnp.float32) + b2_ref[...]
    comp_ref[...] = jax.nn.sigmoid(logit)
    hs_ref[...] = jnp.maximum(
        jnp.dot(x, S1_ref[...], preferred_element_type=jnp.float32) + bs1_ref[...], 0.0)


def _f32_to_key(s):
    b = jax.lax.bitcast_convert_type(s, jnp.int32)
    return jnp.where(b >= 0, b, MAXI32 - b)


def _key_to_f32(k):
    b = jnp.where(k >= 0, k, MAXI32 - k)
    return jax.lax.bitcast_convert_type(b, jnp.float32)


def _keys_bisect_body(hs_ref, S2_ref, bs2_ref, keys_ref, t_ref, quota_ref,
                      lo_ref, hi_ref, fr_ref):
    scores = (jnp.dot(hs_ref[...], S2_ref[...],
                      preferred_element_type=jnp.float32) + bs2_ref[...])
    keys = _f32_to_key(scores)
    keys_ref[...] = keys

    lo_ref[...] = jnp.min(keys, axis=1, keepdims=True)
    hi_ref[...] = jnp.max(keys, axis=1, keepdims=True) + 1
    fr_ref[...] = jnp.zeros(fr_ref.shape, jnp.int32)

    def cond(carry):
        it, done = carry
        return jnp.logical_and(it < 34, done == 0)

    def body(carry):
        it, done = carry
        lo = lo_ref[...]
        hi = hi_ref[...]
        fr = fr_ref[...]
        mid = (lo >> 1) + (hi >> 1) + (lo & hi & 1)
        cnt = jnp.sum((keys >= mid).astype(jnp.int32), axis=1, keepdims=True)
        active = (fr == 0) & (hi - lo != 1)
        ge = cnt >= K
        new_lo = jnp.where(active & ge, mid, lo)
        new_hi = jnp.where(active & jnp.logical_not(ge), mid, hi)
        new_fr = fr | (active & (cnt == K)).astype(jnp.int32)
        lo_ref[...] = new_lo
        hi_ref[...] = new_hi
        fr_ref[...] = new_fr
        all_done = jnp.all((new_fr != 0) | (new_hi - new_lo == 1))
        return it + 1, all_done.astype(jnp.int32)

    jax.lax.while_loop(cond, body, (jnp.int32(0), jnp.int32(0)))

    lo = lo_ref[...]
    frozen = fr_ref[...] != 0
    sel = keys >= lo
    minsel = jnp.min(jnp.where(sel, keys, MAXI32), axis=1, keepdims=True)
    t = jnp.where(frozen, minsel, lo)
    cnt_gt = jnp.sum((keys > t).astype(jnp.int32), axis=1, keepdims=True)
    quota = K - cnt_gt
    t_ref[...] = jnp.broadcast_to(t, t_ref.shape)
    quota_ref[...] = jnp.broadcast_to(quota, quota_ref.shape)


NW = 32
RPW = TOKENS // NW
OPAD = K + 16


def _compact_body(keys_hbm, t_hbm, q_hbm, ck_hbm, ci_hbm,
                  krow, t16, q16, outk, outi):
    c = lax.axis_index("c")
    s = lax.axis_index("s")
    w = s * 2 + c
    row0 = w * RPW
    iota = lax.iota(jnp.int32, 16)

    def row_body(r, _unused):
        row = row0 + r
        pltpu.sync_copy(keys_hbm.at[row], krow)
        pltpu.sync_copy(t_hbm.at[row], t16)
        pltpu.sync_copy(q_hbm.at[row], q16)
        t = t16[...]
        q = q16[...]

        def vbody(j, carry):
            ptr, eqb = carry
            v = krow[pl.ds(j * 16, 16)]
            m_gt = v > t
            m_eq = v == t
            rank = plsc.cumsum(m_eq.astype(jnp.int32)) + eqb
            m_sel = m_gt | (m_eq & (rank <= q))
            plsc.store_compressed(outk.at[pl.ds(ptr, 16)], v, mask=m_sel)
            plsc.store_compressed(outi.at[pl.ds(ptr, 16)], iota + j * 16,
                                  mask=m_sel)
            cnt = plsc.all_reduce_population_count(m_sel)
            eqc = plsc.all_reduce_population_count(m_eq)
            return ptr + jnp.max(cnt), eqb + eqc

        lax.fori_loop(0, POOL_SIZE // 16, vbody,
                      (jnp.int32(0), jnp.zeros((16,), jnp.int32)))
        pltpu.sync_copy(outk.at[pl.ds(0, K)], ck_hbm.at[row])
        pltpu.sync_copy(outi.at[pl.ds(0, K)], ci_hbm.at[row])
        return _unused

    lax.fori_loop(0, RPW, row_body, jnp.int32(0))


def _compact_call(keys, t_b, quota_b):
    mesh = plsc.VectorSubcoreMesh(core_axis_name="c", subcore_axis_name="s", num_cores=2, num_subcores=16)
    f = pl.kernel(
        _compact_body,
        out_type=[
            jax.ShapeDtypeStruct((TOKENS, K), jnp.int32),
            jax.ShapeDtypeStruct((TOKENS, K), jnp.int32),
        ],
        mesh=mesh,
        compiler_params=pltpu.CompilerParams(needs_layout_passes=False),
        scratch_types=[
            pltpu.VMEM((POOL_SIZE,), jnp.int32),
            pltpu.VMEM((16,), jnp.int32),
            pltpu.VMEM((16,), jnp.int32),
            pltpu.VMEM((OPAD,), jnp.int32),
            pltpu.VMEM((OPAD,), jnp.int32),
        ],
    )
    return f(keys, t_b, quota_b)


def _bitonic_sort_desc(keys, idx):
    R, N = keys.shape
    lane = jax.lax.broadcasted_iota(jnp.int32, (R, N), 1)
    k = 2
    while k <= N:
        j = k // 2
        while j >= 1:
            low = (lane & j) == 0
            pk = jnp.where(low, pltpu.roll(keys, N - j, 1), pltpu.roll(keys, j, 1))
            pi = jnp.where(low, pltpu.roll(idx, N - j, 1), pltpu.roll(idx, j, 1))
            self_beats = (keys > pk) | ((keys == pk) & (idx < pi))
            want_max = ((lane & k) == 0) == low
            take_self = want_max == self_beats
            keys = jnp.where(take_self, keys, pk)
            idx = jnp.where(take_self, idx, pi)
            j //= 2
        k *= 2
    return keys, idx


def _sort_finalize_body(ckeys_ref, cidx_ref, comp_ref,
                        out_idx_ref, out_w_ref, out_mask_ref):
    keys, idx = _bitonic_sort_desc(ckeys_ref[...], cidx_ref[...])
    ts = _key_to_f32(keys)
    m = ts[:, 0:1]
    e = jnp.exp(ts - m)
    w = e / jnp.sum(e, axis=1, keepdims=True)
    comp = comp_ref[...]
    budgets = jnp.round(
        jnp.clip(MIN_P + (MAX_P - MIN_P) * comp * comp, MIN_P, MAX_P)
    ).astype(jnp.int32)
    pos = jax.lax.broadcasted_iota(jnp.int32, out_mask_ref.shape, 1)
    mask = (pos < budgets).astype(jnp.float32)
    out_idx_ref[...] = idx
    out_w_ref[...] = w * mask
    out_mask_ref[...] = mask


def kernel(x, W1, b1, W2, b2, S1, bs1, S2, bs2):
    n_ra = TOKENS // ROW_A
    hs, comp = pl.pallas_call(
        _hs_complexity_body,
        grid=(n_ra,),
        in_specs=[
            pl.BlockSpec((ROW_A, INPUT_DIM), lambda i: (i, 0)),
            pl.BlockSpec((INPUT_DIM, 128), lambda i: (0, 0)),
            pl.BlockSpec((128,), lambda i: (0,)),
            pl.BlockSpec((128, 1), lambda i: (0, 0)),
            pl.BlockSpec((1,), lambda i: (0,)),
            pl.BlockSpec((INPUT_DIM, HIDDEN_DIM), lambda i: (0, 0)),
            pl.BlockSpec((HIDDEN_DIM,), lambda i: (0,)),
        ],
        out_specs=[
            pl.BlockSpec((ROW_A, HIDDEN_DIM), lambda i: (i, 0)),
            pl.BlockSpec((ROW_A, 1), lambda i: (i, 0)),
        ],
        out_shape=[
            jax.ShapeDtypeStruct((TOKENS, HIDDEN_DIM), jnp.float32),
            jax.ShapeDtypeStruct((TOKENS, 1), jnp.float32),
        ],
    )(x, W1, b1, W2, b2, S1, bs1)

    n_rb = TOKENS // ROW_B
    keys, t_b, quota_b = pl.pallas_call(
        _keys_bisect_body,
        grid=(n_rb,),
        in_specs=[
            pl.BlockSpec((ROW_B, HIDDEN_DIM), lambda i: (i, 0)),
            pl.BlockSpec((HIDDEN_DIM, POOL_SIZE), lambda i: (0, 0)),
            pl.BlockSpec((POOL_SIZE,), lambda i: (0,)),
        ],
        out_specs=[
            pl.BlockSpec((ROW_B, POOL_SIZE), lambda i: (i, 0)),
            pl.BlockSpec((ROW_B, 16), lambda i: (i, 0)),
            pl.BlockSpec((ROW_B, 16), lambda i: (i, 0)),
        ],
        out_shape=[
            jax.ShapeDtypeStruct((TOKENS, POOL_SIZE), jnp.int32),
            jax.ShapeDtypeStruct((TOKENS, 16), jnp.int32),
            jax.ShapeDtypeStruct((TOKENS, 16), jnp.int32),
        ],
        scratch_shapes=[
            pltpu.VMEM((ROW_B, 1), jnp.int32),
            pltpu.VMEM((ROW_B, 1), jnp.int32),
            pltpu.VMEM((ROW_B, 1), jnp.int32),
        ],
    )(hs, S2, bs2)

    ckeys, cidx = _compact_call(keys, t_b, quota_b)

    n_rd = TOKENS // ROW_D
    out_idx, out_w, out_mask = pl.pallas_call(
        _sort_finalize_body,
        grid=(n_rd,),
        in_specs=[
            pl.BlockSpec((ROW_D, K), lambda i: (i, 0)),
            pl.BlockSpec((ROW_D, K), lambda i: (i, 0)),
            pl.BlockSpec((ROW_D, 1), lambda i: (i, 0)),
        ],
        out_specs=[
            pl.BlockSpec((ROW_D, K), lambda i: (i, 0)),
            pl.BlockSpec((ROW_D, K), lambda i: (i, 0)),
            pl.BlockSpec((ROW_D, K), lambda i: (i, 0)),
        ],
        out_shape=[
            jax.ShapeDtypeStruct((TOKENS, K), jnp.int32),
            jax.ShapeDtypeStruct((TOKENS, K), jnp.float32),
            jax.ShapeDtypeStruct((TOKENS, K), jnp.float32),
        ],
    )(ckeys, cidx, comp)
    return out_idx, out_w, out_mask, comp
